# baseline (speedup 1.0000x reference)
import jax
import jax.numpy as jnp
from jax.experimental import pallas as pl
from jax.experimental.pallas import tpu as pltpu

_EPS = 1e-10
_PAD = -1.0
_ROW_BLOCK = 128
_I_CHUNK = 64


def _listmle_block(yt_ref, yp_ref, out_ref, e_ref):
    yt = yt_ref[...]
    yp = yp_ref[...]
    n = yt.shape[0]
    valid = yt != _PAD
    pmask = jnp.where(valid, yp, -jnp.inf)
    m = jnp.max(pmask, axis=0, keepdims=True)
    e = jnp.where(valid, jnp.exp(yp - m), 0.0)
    e_ref[...] = e
    s = jnp.sum(e, axis=0, keepdims=True)

    obs_total = jnp.zeros(m.shape, jnp.float32)
    for c in range(n // _I_CHUNK):
        yt_c = jax.lax.slice_in_dim(yt, c * _I_CHUNK, (c + 1) * _I_CHUNK, axis=0)

        def body(j, carry, yt_c=yt_c):
            r, a = carry
            yj = yt_ref[pl.ds(j, 1), :]
            ej = e_ref[pl.ds(j, 1), :]
            f = (yj > yt_c).astype(jnp.float32)
            return (r + f, a + f * ej)

        zero = jnp.zeros((_I_CHUNK, yt.shape[1]), jnp.float32)
        r, a = jax.lax.fori_loop(0, n, body, (zero, zero))
        cum = s - a
        yp_c = jax.lax.slice_in_dim(yp, c * _I_CHUNK, (c + 1) * _I_CHUNK, axis=0)
        valid_c = jax.lax.slice_in_dim(valid, c * _I_CHUNK, (c + 1) * _I_CHUNK, axis=0)
        obs = (jnp.log(cum + _EPS) - (yp_c - m)) / jnp.log(r + 2.0)
        obs = jnp.where(valid_c, obs, 0.0)
        obs_total = obs_total + jnp.sum(obs, axis=0, keepdims=True)
    out_ref[...] = obs_total


def _row_losses(y_pred, y_true, interpret=False):
    b, n = y_pred.shape
    ypt = y_pred.T
    ytt = y_true.T
    rb = min(_ROW_BLOCK, b)
    return pl.pallas_call(
        _listmle_block,
        grid=(b // rb,),
        in_specs=[
            pl.BlockSpec((n, rb), lambda i: (0, i)),
            pl.BlockSpec((n, rb), lambda i: (0, i)),
        ],
        out_specs=pl.BlockSpec((1, rb), lambda i: (0, i)),
        out_shape=jax.ShapeDtypeStruct((1, b), jnp.float32),
        scratch_shapes=[pltpu.VMEM((n, rb), jnp.float32)],
        interpret=interpret,
    )(ytt, ypt)


def kernel(y_pred, y_true):
    y_pred = y_pred.reshape(-1, y_pred.shape[-1])
    y_true = y_true.reshape(-1, y_true.shape[-1])
    row = _row_losses(y_pred, y_true)
    return jnp.mean(row)


# speedup vs baseline: 7.0878x; 7.0878x over previous
import jax
import jax.numpy as jnp
from jax import lax
from jax.experimental import pallas as pl
from jax.experimental.pallas import tpu as pltpu

_EPS = 1e-10
_PAD = -1.0
_ROW_BLOCK = 128


def _rot(x, sh):
    return jnp.concatenate([x[sh:], x[:sh]], axis=0)


def _listmle_block(yt_ref, yp_ref, out_ref):
    key = yt_ref[...]
    val = yp_ref[...]
    n, rb = key.shape
    ii = lax.broadcasted_iota(jnp.int32, (n, rb), 0)
    ii1 = lax.broadcasted_iota(jnp.int32, (n, 1), 0)

    kblk = 2
    while kblk <= n:
        d = kblk // 2
        while d >= 1:
            am_lo = (ii & d) == 0
            takemax = am_lo == ((ii & kblk) == 0)
            kp = jnp.where(am_lo, _rot(key, d), _rot(key, -d))
            vp = jnp.where(am_lo, _rot(val, d), _rot(val, -d))
            took = (takemax & (kp > key)) | (~takemax & (kp < key))
            key = jnp.where(took, kp, key)
            val = jnp.where(took, vp, val)
            d //= 2
        kblk *= 2

    valid = key != _PAD
    pm = jnp.where(valid, val, -jnp.inf)
    m = jnp.max(pm, axis=0, keepdims=True)
    e = jnp.where(valid, jnp.exp(val - m), 0.0)

    c = e
    sh = 1
    while sh < n:
        c = c + jnp.concatenate([c[sh:], jnp.zeros((sh, c.shape[1]), c.dtype)], axis=0)
        sh *= 2

    w = jnp.log(ii1.astype(jnp.float32) + 2.0)
    obs = (jnp.log(c + _EPS) - (val - m)) / w
    obs = jnp.where(valid, obs, 0.0)
    out_ref[...] = jnp.sum(obs, axis=0, keepdims=True)


def _row_losses(y_pred, y_true, interpret=False):
    b, n = y_pred.shape
    ypt = y_pred.T
    ytt = y_true.T
    rb = min(_ROW_BLOCK, b)
    return pl.pallas_call(
        _listmle_block,
        grid=(b // rb,),
        in_specs=[
            pl.BlockSpec((n, rb), lambda i: (0, i)),
            pl.BlockSpec((n, rb), lambda i: (0, i)),
        ],
        out_specs=pl.BlockSpec((1, rb), lambda i: (0, i)),
        out_shape=jax.ShapeDtypeStruct((1, b), jnp.float32),
        interpret=interpret,
    )(ytt, ypt)


def kernel(y_pred, y_true):
    y_pred = y_pred.reshape(-1, y_pred.shape[-1])
    y_true = y_true.reshape(-1, y_true.shape[-1])
    row = _row_losses(y_pred, y_true)
    return jnp.mean(row)
